# baseline (speedup 1.0000x reference)
import functools

import jax
import jax.numpy as jnp
from jax import lax
from jax.experimental import pallas as pl
from jax.experimental.pallas import tpu as pltpu
from jax.experimental.pallas import tpu_sc as plsc

DIM = 64
L = 16
WAVE = 16


def _sc_partial_scores(ent1_re, ent1_im, rel1_re, rel1_im, h_idx, r_idx, t_idx):
    b_total = h_idx.shape[0]
    rel_words = rel1_re.shape[0]
    info = plsc.get_sparse_core_info()
    nw = info.num_cores * info.num_subcores
    chunk = b_total // nw
    nwave = chunk // WAVE
    npair = nwave // 2
    assert chunk % (2 * WAVE) == 0

    mesh = plsc.VectorSubcoreMesh(core_axis_name="c", subcore_axis_name="s")

    buf_t = pltpu.VMEM((WAVE * DIM,), jnp.float32)

    @functools.partial(
        pl.kernel,
        mesh=mesh,
        out_type=jax.ShapeDtypeStruct((b_total * L,), jnp.float32),
        scratch_types=[
            pltpu.VMEM((chunk,), jnp.int32),
            pltpu.VMEM((chunk,), jnp.int32),
            pltpu.VMEM((chunk,), jnp.int32),
            [buf_t] * 6,
            pltpu.VMEM((chunk * L,), jnp.float32),
            pltpu.VMEM_SHARED((rel_words,), jnp.float32),
            pltpu.VMEM_SHARED((rel_words,), jnp.float32),
            pltpu.SemaphoreType.DMA,
        ],
    )
    def sc_kernel(ent_re_hbm, ent_im_hbm, rel_re_hbm, rel_im_hbm,
                  h_hbm, r_hbm, t_hbm, out_hbm,
                  h_v, r_v, t_v, bufs, part_v, relre_sp, relim_sp,
                  sem):
        cid = lax.axis_index("c")
        sid = lax.axis_index("s")
        wid = sid * info.num_cores + cid
        base = wid * chunk

        @pl.when(sid == 0)
        def _stage_rel():
            rchunk = rel_words // 8
            bounce = part_v.at[pl.ds(0, rchunk)]

            def stage_body(j, carry):
                sl = pl.ds(j * rchunk, rchunk)
                pltpu.sync_copy(rel_re_hbm.at[sl], bounce)
                pltpu.sync_copy(bounce, relre_sp.at[sl])
                pltpu.sync_copy(rel_im_hbm.at[sl], bounce)
                pltpu.sync_copy(bounce, relim_sp.at[sl])
                return carry

            lax.fori_loop(0, 8, stage_body, 0)

        pltpu.sync_copy(h_hbm.at[pl.ds(base, chunk)], h_v)
        pltpu.sync_copy(r_hbm.at[pl.ds(base, chunk)], r_v)
        pltpu.sync_copy(t_hbm.at[pl.ds(base, chunk)], t_v)
        plsc.subcore_barrier()

        hre_b, him_b, tre_b, tim_b, rre_b, rim_b = bufs

        def wave_body(w, carry):
            row0 = w * WAVE
            hv16 = h_v[pl.ds(row0, L)]
            tv16 = t_v[pl.ds(row0, L)]
            rv16 = r_v[pl.ds(row0, L)]
            cps = []
            for g in range(WAVE):
                ih = hv16[g] * DIM
                it = tv16[g] * DIM
                ir = rv16[g] * DIM
                dst = pl.ds(g * DIM, DIM)
                cps.append(pltpu.async_copy(ent_re_hbm.at[pl.ds(ih, DIM)], hre_b.at[dst], sem))
                cps.append(pltpu.async_copy(ent_im_hbm.at[pl.ds(ih, DIM)], him_b.at[dst], sem))
                cps.append(pltpu.async_copy(ent_re_hbm.at[pl.ds(it, DIM)], tre_b.at[dst], sem))
                cps.append(pltpu.async_copy(ent_im_hbm.at[pl.ds(it, DIM)], tim_b.at[dst], sem))
                cps.append(pltpu.async_copy(relre_sp.at[pl.ds(ir, DIM)], rre_b.at[dst], sem))
                cps.append(pltpu.async_copy(relim_sp.at[pl.ds(ir, DIM)], rim_b.at[dst], sem))
            for cp in cps:
                cp.wait()
            for g in range(WAVE):
                acc = jnp.zeros((L,), jnp.float32)
                for c in range(DIM // L):
                    sl = pl.ds(g * DIM + c * L, L)
                    hre = hre_b[sl]
                    him = him_b[sl]
                    tre = tre_b[sl]
                    tim = tim_b[sl]
                    rre = rre_b[sl]
                    rim = rim_b[sl]
                    acc = acc + rre * (hre * tre + him * tim) + rim * (hre * tim - him * tre)
                part_v[pl.ds((row0 + g) * L, L)] = acc
            return carry

        lax.fori_loop(0, nwave, wave_body, 0)
        pltpu.sync_copy(part_v, out_hbm.at[pl.ds(base * L, chunk * L)])

    return sc_kernel(ent1_re, ent1_im, rel1_re, rel1_im, h_idx, r_idx, t_idx)


def _loss_tc_kernel(part_ref, out_ref):
    x = part_ref[...]
    s = jnp.sum(x, axis=2)
    sgn = jnp.concatenate(
        [jnp.full((1, s.shape[1]), -1.0, jnp.float32),
         jnp.full((1, s.shape[1]), 1.0, jnp.float32)], axis=0)
    z = s * sgn
    sp = jnp.maximum(z, 0.0) + jnp.log1p(jnp.exp(-jnp.abs(z)))
    out_ref[...] = jnp.mean(sp, axis=(0, 1), keepdims=True).reshape(1, 1)


def kernel(ent_re, ent_im, rel_re, rel_im, positive_triples, negative_triples):
    b = positive_triples.shape[0]
    h_idx = jnp.concatenate(
        [positive_triples[:, 0], negative_triples[:, 0]]).astype(jnp.int32)
    r_idx = jnp.concatenate(
        [positive_triples[:, 1], negative_triples[:, 1]]).astype(jnp.int32)
    t_idx = jnp.concatenate(
        [positive_triples[:, 2], negative_triples[:, 2]]).astype(jnp.int32)

    part = _sc_partial_scores(
        ent_re.reshape(-1), ent_im.reshape(-1),
        rel_re.reshape(-1), rel_im.reshape(-1),
        h_idx, r_idx, t_idx)
    part3 = part.reshape(2, b, L)

    loss = pl.pallas_call(
        _loss_tc_kernel,
        out_shape=jax.ShapeDtypeStruct((1, 1), jnp.float32),
    )(part3)
    return loss.reshape(())


# speedup vs baseline: 1.3363x; 1.3363x over previous
import functools

import jax
import jax.numpy as jnp
from jax import lax
from jax.experimental import pallas as pl
from jax.experimental.pallas import tpu as pltpu
from jax.experimental.pallas import tpu_sc as plsc

DIM = 64
L = 16
GRP = 8
WAVE = 8


def _sc_partial_scores(ent3_re, ent3_im, rel3_re, rel3_im, h_idx, r_idx, t_idx):
    b_total = h_idx.shape[0]
    nrel_grp = rel3_re.shape[0]
    info = plsc.get_sparse_core_info()
    nw = info.num_cores * info.num_subcores
    chunk = b_total // nw
    nwave = chunk // WAVE
    npair = nwave // 2
    assert chunk % (2 * WAVE) == 0

    mesh = plsc.VectorSubcoreMesh(core_axis_name="c", subcore_axis_name="s")

    @functools.partial(
        pl.kernel,
        mesh=mesh,
        out_type=jax.ShapeDtypeStruct((b_total * L,), jnp.float32),
        scratch_types=[
            pltpu.VMEM((chunk,), jnp.int32),
            pltpu.VMEM((chunk,), jnp.int32),
            pltpu.VMEM((chunk,), jnp.int32),
            [[pltpu.VMEM((WAVE, GRP, DIM), jnp.float32)] * 4
             + [pltpu.VMEM((WAVE, DIM), jnp.float32)] * 2] * 2,
            pltpu.VMEM((chunk * L,), jnp.float32),
            pltpu.VMEM_SHARED((nrel_grp, GRP, DIM), jnp.float32),
            pltpu.VMEM_SHARED((nrel_grp, GRP, DIM), jnp.float32),
            pltpu.SemaphoreType.DMA,
            pltpu.SemaphoreType.DMA,
        ],
    )
    def sc_kernel(ent_re_hbm, ent_im_hbm, rel_re_hbm, rel_im_hbm,
                  h_hbm, r_hbm, t_hbm, out_hbm,
                  h_v, r_v, t_v, bufs,
                  part_v, relre_sp, relim_sp, sem0, sem1):
        cid = lax.axis_index("c")
        sid = lax.axis_index("s")
        wid = sid * info.num_cores + cid
        base = wid * chunk

        @pl.when(sid == 0)
        def _stage_rel():
            def stage_body(j, carry):
                pltpu.sync_copy(rel_re_hbm.at[j], relre_sp.at[j])
                pltpu.sync_copy(rel_im_hbm.at[j], relim_sp.at[j])
                return carry
            lax.fori_loop(0, nrel_grp, stage_body, 0)

        pltpu.sync_copy(h_hbm.at[pl.ds(base, chunk)], h_v)
        pltpu.sync_copy(r_hbm.at[pl.ds(base, chunk)], r_v)
        pltpu.sync_copy(t_hbm.at[pl.ds(base, chunk)], t_v)
        plsc.subcore_barrier()
        sems = [sem0, sem1]

        def fire(w, bset):
            row0 = w * WAVE
            hv = h_v[pl.ds(row0, L)]
            tv = t_v[pl.ds(row0, L)]
            rv = r_v[pl.ds(row0, L)]
            hre_b, him_b, tre_b, tim_b, rre_b, rim_b = bufs[bset]
            sem = sems[bset]
            cps = []
            for g in range(WAVE):
                ih = hv[g]
                it = tv[g]
                ir = rv[g]
                cps.append(pltpu.async_copy(
                    ent_re_hbm.at[lax.shift_right_logical(ih, 3)], hre_b.at[g], sem))
                cps.append(pltpu.async_copy(
                    ent_im_hbm.at[lax.shift_right_logical(ih, 3)], him_b.at[g], sem))
                cps.append(pltpu.async_copy(
                    ent_re_hbm.at[lax.shift_right_logical(it, 3)], tre_b.at[g], sem))
                cps.append(pltpu.async_copy(
                    ent_im_hbm.at[lax.shift_right_logical(it, 3)], tim_b.at[g], sem))
                cps.append(pltpu.async_copy(
                    relre_sp.at[lax.shift_right_logical(ir, 3), ir & 7], rre_b.at[g], sem))
                cps.append(pltpu.async_copy(
                    relim_sp.at[lax.shift_right_logical(ir, 3), ir & 7], rim_b.at[g], sem))
            return cps

        def compute(w, bset):
            row0 = w * WAVE
            hv = h_v[pl.ds(row0, L)]
            tv = t_v[pl.ds(row0, L)]
            hre_b, him_b, tre_b, tim_b, rre_b, rim_b = bufs[bset]
            for g in range(WAVE):
                rh = hv[g] & 7
                rt = tv[g] & 7
                acc = jnp.zeros((L,), jnp.float32)
                for c in range(DIM // L):
                    sl = pl.ds(c * L, L)
                    hre = hre_b[g, rh, sl]
                    him = him_b[g, rh, sl]
                    tre = tre_b[g, rt, sl]
                    tim = tim_b[g, rt, sl]
                    rre = rre_b[g, sl]
                    rim = rim_b[g, sl]
                    acc = acc + rre * (hre * tre + him * tim) + rim * (hre * tim - him * tre)
                part_v[pl.ds((row0 + g) * L, L)] = acc

        def pair_body(p, carry):
            w0 = p * 2
            cps0 = fire(w0, 0)
            cps1 = fire(w0 + 1, 1)
            for cp in cps0:
                cp.wait()
            compute(w0, 0)
            for cp in cps1:
                cp.wait()
            compute(w0 + 1, 1)
            return carry

        lax.fori_loop(0, npair, pair_body, 0)
        pltpu.sync_copy(part_v, out_hbm.at[pl.ds(base * L, chunk * L)])

    return sc_kernel(ent3_re, ent3_im, rel3_re, rel3_im, h_idx, r_idx, t_idx)


def _loss_tc_kernel(part_ref, out_ref):
    x = part_ref[...]
    s = jnp.sum(x, axis=2)
    sgn = jnp.concatenate(
        [jnp.full((1, s.shape[1]), -1.0, jnp.float32),
         jnp.full((1, s.shape[1]), 1.0, jnp.float32)], axis=0)
    z = s * sgn
    sp = jnp.maximum(z, 0.0) + jnp.log1p(jnp.exp(-jnp.abs(z)))
    out_ref[...] = jnp.mean(sp, axis=(0, 1), keepdims=True).reshape(1, 1)


def kernel(ent_re, ent_im, rel_re, rel_im, positive_triples, negative_triples):
    b = positive_triples.shape[0]
    h_idx = jnp.concatenate(
        [positive_triples[:, 0], negative_triples[:, 0]]).astype(jnp.int32)
    r_idx = jnp.concatenate(
        [positive_triples[:, 1], negative_triples[:, 1]]).astype(jnp.int32)
    t_idx = jnp.concatenate(
        [positive_triples[:, 2], negative_triples[:, 2]]).astype(jnp.int32)

    ent3_re = ent_re.reshape(-1, GRP, DIM)
    ent3_im = ent_im.reshape(-1, GRP, DIM)
    rel3_re = rel_re.reshape(-1, GRP, DIM)
    rel3_im = rel_im.reshape(-1, GRP, DIM)

    part = _sc_partial_scores(ent3_re, ent3_im, rel3_re, rel3_im,
                              h_idx, r_idx, t_idx)
    part3 = part.reshape(2, b, L)

    loss = pl.pallas_call(
        _loss_tc_kernel,
        out_shape=jax.ShapeDtypeStruct((1, 1), jnp.float32),
    )(part3)
    return loss.reshape(())


# speedup vs baseline: 1.3553x; 1.0142x over previous
import functools

import jax
import jax.numpy as jnp
from jax import lax
from jax.experimental import pallas as pl
from jax.experimental.pallas import tpu as pltpu
from jax.experimental.pallas import tpu_sc as plsc

DIM = 64
L = 16
GRP = 8
WAVE = 16


def _sc_partial_scores(ent3_re, ent3_im, rel3_re, rel3_im, h_idx, r_idx, t_idx):
    b_total = h_idx.shape[0]
    nrel_grp = rel3_re.shape[0]
    info = plsc.get_sparse_core_info()
    nw = info.num_cores * info.num_subcores
    chunk = b_total // nw
    nwave = chunk // WAVE
    assert chunk % WAVE == 0

    mesh = plsc.VectorSubcoreMesh(core_axis_name="c", subcore_axis_name="s")

    @functools.partial(
        pl.kernel,
        mesh=mesh,
        out_type=jax.ShapeDtypeStruct((b_total * L,), jnp.float32),
        scratch_types=[
            pltpu.VMEM((chunk,), jnp.int32),
            pltpu.VMEM((chunk,), jnp.int32),
            pltpu.VMEM((chunk,), jnp.int32),
            pltpu.VMEM((WAVE, GRP, DIM), jnp.float32),
            pltpu.VMEM((WAVE, GRP, DIM), jnp.float32),
            pltpu.VMEM((WAVE, GRP, DIM), jnp.float32),
            pltpu.VMEM((WAVE, GRP, DIM), jnp.float32),
            pltpu.VMEM((WAVE, DIM), jnp.float32),
            pltpu.VMEM((WAVE, DIM), jnp.float32),
            pltpu.VMEM((chunk * L,), jnp.float32),
            pltpu.VMEM_SHARED((nrel_grp, GRP, DIM), jnp.float32),
            pltpu.VMEM_SHARED((nrel_grp, GRP, DIM), jnp.float32),
            pltpu.SemaphoreType.DMA,
        ],
    )
    def sc_kernel(ent_re_hbm, ent_im_hbm, rel_re_hbm, rel_im_hbm,
                  h_hbm, r_hbm, t_hbm, out_hbm,
                  h_v, r_v, t_v, hre_v, him_v, tre_v, tim_v, rre_v, rim_v,
                  part_v, relre_sp, relim_sp, sem):
        cid = lax.axis_index("c")
        sid = lax.axis_index("s")
        wid = sid * info.num_cores + cid
        base = wid * chunk

        @pl.when(sid == 0)
        def _stage_rel():
            def stage_body(j, carry):
                pltpu.sync_copy(rel_re_hbm.at[j], relre_sp.at[j])
                pltpu.sync_copy(rel_im_hbm.at[j], relim_sp.at[j])
                return carry
            lax.fori_loop(0, nrel_grp, stage_body, 0)

        pltpu.sync_copy(h_hbm.at[pl.ds(base, chunk)], h_v)
        pltpu.sync_copy(r_hbm.at[pl.ds(base, chunk)], r_v)
        pltpu.sync_copy(t_hbm.at[pl.ds(base, chunk)], t_v)
        plsc.subcore_barrier()

        def wave_body(w, carry):
            row0 = w * WAVE
            hv16 = h_v[pl.ds(row0, L)]
            tv16 = t_v[pl.ds(row0, L)]
            rv16 = r_v[pl.ds(row0, L)]
            cps = []
            for g in range(WAVE):
                ih = hv16[g]
                it = tv16[g]
                ir = rv16[g]
                cps.append(pltpu.async_copy(
                    ent_re_hbm.at[lax.shift_right_logical(ih, 3)], hre_v.at[g], sem))
                cps.append(pltpu.async_copy(
                    ent_im_hbm.at[lax.shift_right_logical(ih, 3)], him_v.at[g], sem))
                cps.append(pltpu.async_copy(
                    ent_re_hbm.at[lax.shift_right_logical(it, 3)], tre_v.at[g], sem))
                cps.append(pltpu.async_copy(
                    ent_im_hbm.at[lax.shift_right_logical(it, 3)], tim_v.at[g], sem))
                cps.append(pltpu.async_copy(
                    relre_sp.at[lax.shift_right_logical(ir, 3), ir & 7], rre_v.at[g], sem))
                cps.append(pltpu.async_copy(
                    relim_sp.at[lax.shift_right_logical(ir, 3), ir & 7], rim_v.at[g], sem))
            for cp in cps:
                cp.wait()
            for g in range(WAVE):
                rh = hv16[g] & 7
                rt = tv16[g] & 7
                acc = jnp.zeros((L,), jnp.float32)
                for c in range(DIM // L):
                    sl = pl.ds(c * L, L)
                    hre = hre_v[g, rh, sl]
                    him = him_v[g, rh, sl]
                    tre = tre_v[g, rt, sl]
                    tim = tim_v[g, rt, sl]
                    rre = rre_v[g, sl]
                    rim = rim_v[g, sl]
                    acc = acc + rre * (hre * tre + him * tim) + rim * (hre * tim - him * tre)
                part_v[pl.ds((row0 + g) * L, L)] = acc
            return carry

        lax.fori_loop(0, nwave, wave_body, 0)
        pltpu.sync_copy(part_v, out_hbm.at[pl.ds(base * L, chunk * L)])

    return sc_kernel(ent3_re, ent3_im, rel3_re, rel3_im, h_idx, r_idx, t_idx)


def _loss_tc_kernel(part_ref, out_ref):
    x = part_ref[...]
    s = jnp.sum(x, axis=2)
    sgn = jnp.concatenate(
        [jnp.full((1, s.shape[1]), -1.0, jnp.float32),
         jnp.full((1, s.shape[1]), 1.0, jnp.float32)], axis=0)
    z = s * sgn
    sp = jnp.maximum(z, 0.0) + jnp.log1p(jnp.exp(-jnp.abs(z)))
    out_ref[...] = jnp.mean(sp, axis=(0, 1), keepdims=True).reshape(1, 1)


def kernel(ent_re, ent_im, rel_re, rel_im, positive_triples, negative_triples):
    b = positive_triples.shape[0]
    h_idx = jnp.concatenate(
        [positive_triples[:, 0], negative_triples[:, 0]]).astype(jnp.int32)
    r_idx = jnp.concatenate(
        [positive_triples[:, 1], negative_triples[:, 1]]).astype(jnp.int32)
    t_idx = jnp.concatenate(
        [positive_triples[:, 2], negative_triples[:, 2]]).astype(jnp.int32)

    ent3_re = ent_re.reshape(-1, GRP, DIM)
    ent3_im = ent_im.reshape(-1, GRP, DIM)
    rel3_re = rel_re.reshape(-1, GRP, DIM)
    rel3_im = rel_im.reshape(-1, GRP, DIM)

    part = _sc_partial_scores(ent3_re, ent3_im, rel3_re, rel3_im,
                              h_idx, r_idx, t_idx)
    part3 = part.reshape(2, b, L)

    loss = pl.pallas_call(
        _loss_tc_kernel,
        out_shape=jax.ShapeDtypeStruct((1, 1), jnp.float32),
    )(part3)
    return loss.reshape(())
